# baseline (speedup 1.0000x reference)
import functools

import jax
import jax.numpy as jnp
from jax import lax
from jax.experimental import pallas as pl
from jax.experimental.pallas import tpu as pltpu
from jax.experimental.pallas import tpu_sc as plsc

_NPIX = 196608
_NFREQ = 128
_NPTS = 65536

_NUM_CORES = 2
_NUM_SUBCORES = 16
_NUM_WORKERS = _NUM_CORES * _NUM_SUBCORES
_LANES = 16
_CHUNK_PTS = 32
_NBUF = 4
_GROUP_PTS = 128
_NOUT = 2
_NWRD = _NFREQ // 2


def _pixel_beam_sc(table, idx2d, wab):
    ppw = _NPTS // _NUM_WORKERS
    chunks = ppw // _CHUNK_PTS
    groups = ppw // _GROUP_PTS
    cpg = _GROUP_PTS // _CHUNK_PTS
    rows_pc = _CHUNK_PTS * 4
    nblk = _NWRD // _LANES

    mesh = plsc.VectorSubcoreMesh(core_axis_name="c", subcore_axis_name="s")

    @functools.partial(
        pl.kernel,
        out_type=jax.ShapeDtypeStruct((_NPTS, _NFREQ), jnp.float32),
        mesh=mesh,
        compiler_params=pltpu.CompilerParams(needs_layout_passes=False),
        scratch_types=[
            pltpu.VMEM((chunks, 128), jnp.int32),
            pltpu.VMEM((ppw * 8,), jnp.float32),
            pltpu.VMEM((_NBUF, rows_pc, 2 * _NWRD), jnp.int32),
            pltpu.VMEM((_NOUT, _GROUP_PTS, _NFREQ), jnp.float32),
            pltpu.SemaphoreType.DMA,
            pltpu.SemaphoreType.DMA,
        ],
    )
    def sc_kernel(table_h, idx_h, wab_h, out_h, idx_v, wgt_v, rows_v, outb,
                  gsem, fsem):
        wid = lax.axis_index("s") * _NUM_CORES + lax.axis_index("c")
        pltpu.sync_copy(idx_h.at[pl.ds(wid * chunks, chunks)], idx_v)
        pltpu.sync_copy(wab_h.at[pl.ds(wid * ppw * 8, ppw * 8)], wgt_v)
        himask = jnp.full((_LANES,), -65536, jnp.int32)

        def gather(c):
            return pltpu.async_copy(
                table_h.at[idx_v.at[c]], rows_v.at[c % _NBUF], gsem)

        for c in range(_NBUF - 1):
            gather(c)

        def flush_copy(g):
            gstart = wid * ppw + g * _GROUP_PTS
            return pltpu.make_async_copy(
                outb.at[g % _NOUT], out_h.at[pl.ds(gstart, _GROUP_PTS)], fsem)

        def group_body(g, carry):
            @pl.when(g >= _NOUT)
            def _drain():
                flush_copy(g - _NOUT).wait()

            ob = outb.at[g % _NOUT]
            for cc in range(cpg):
                c = g * cpg + cc
                pltpu.make_async_copy(
                    table_h.at[idx_v.at[c]], rows_v.at[c % _NBUF], gsem).wait()

                @pl.when(c + _NBUF - 1 < chunks)
                def _prefetch():
                    gather(c + _NBUF - 1)

                rows = rows_v.at[c % _NBUF]

                @plsc.parallel_loop(0, _CHUNK_PTS, 1, unroll=4)
                def _pts(pp):
                    wof = 8 * (c * _CHUNK_PTS + pp)
                    acc_lo = [None] * nblk
                    acc_hi = [None] * nblk
                    for k in range(4):
                        wa = plsc.load_gather(
                            wgt_v,
                            [jnp.full((_LANES,), wof + 2 * k, jnp.int32)])
                        wb = plsc.load_gather(
                            wgt_v,
                            [jnp.full((_LANES,), wof + 2 * k + 1, jnp.int32)])
                        r = 4 * pp + k
                        for j in range(nblk):
                            va = rows[r, pl.ds(j * _LANES, _LANES)]
                            vb = rows[r, pl.ds(_NWRD + j * _LANES, _LANES)]
                            lo = (wa * plsc.bitcast(va << 16, jnp.float32)
                                  + wb * plsc.bitcast(vb << 16, jnp.float32))
                            hi = (wa * plsc.bitcast(va & himask, jnp.float32)
                                  + wb * plsc.bitcast(vb & himask, jnp.float32))
                            if k == 0:
                                acc_lo[j] = lo
                                acc_hi[j] = hi
                            else:
                                acc_lo[j] = acc_lo[j] + lo
                                acc_hi[j] = acc_hi[j] + hi
                    prow = cc * _CHUNK_PTS + pp
                    for j in range(nblk):
                        ob[prow, pl.ds(j * _LANES, _LANES)] = acc_lo[j]
                        ob[prow, pl.ds(_NWRD + j * _LANES, _LANES)] = acc_hi[j]

            flush_copy(g).start()
            return carry

        lax.fori_loop(0, groups, group_body, 0)
        for g in range(groups - _NOUT, groups):
            flush_copy(g).wait()

    return sc_kernel(table, idx2d, wab)


def kernel(params, inds, wgts, freqs):
    pb = params.reshape(_NFREQ, _NPIX).astype(jnp.bfloat16)
    z = pb.reshape(2, _NWRD, _NPIX // 2, 2)
    t4 = z.transpose(2, 3, 1, 0)
    table = lax.bitcast_convert_type(
        t4.reshape(_NPIX // 2, _NFREQ, 2), jnp.int32)
    inds32 = inds.astype(jnp.int32)
    idx2d = (inds32 >> 1).reshape(_NPTS * 4 // 128, 128)
    par = (inds32 & 1).astype(jnp.float32)
    w = wgts.astype(jnp.float32)
    wab = jnp.stack([w * (1.0 - par), w * par], axis=-1).reshape(_NPTS * 8)
    out_pt = _pixel_beam_sc(table, idx2d, wab)
    return out_pt.T.reshape(1, 1, 1, _NFREQ, _NPTS)


# speedup vs baseline: 5.8336x; 5.8336x over previous
import functools

import jax
import jax.numpy as jnp
from jax import lax
from jax.experimental import pallas as pl
from jax.experimental.pallas import tpu as pltpu
from jax.experimental.pallas import tpu_sc as plsc

_NPIX = 196608
_NFREQ = 128
_NPTS = 65536

_NUM_CORES = 2
_NUM_SUBCORES = 16
_NUM_WORKERS = _NUM_CORES * _NUM_SUBCORES
_LANES = 16
_CHUNK_PTS = 32
_NBUF = 4
_GROUP_PTS = 128
_NOUT = 2


def _pixel_beam_sc(table, idx2d, wgt):
    ppw = _NPTS // _NUM_WORKERS
    chunks = ppw // _CHUNK_PTS
    groups = ppw // _GROUP_PTS
    cpg = _GROUP_PTS // _CHUNK_PTS
    rows_pc = _CHUNK_PTS * 4
    nblk = _NFREQ // _LANES

    mesh = plsc.VectorSubcoreMesh(core_axis_name="c", subcore_axis_name="s")

    @functools.partial(
        pl.kernel,
        out_type=jax.ShapeDtypeStruct((_NPTS, _NFREQ), jnp.float32),
        mesh=mesh,
        compiler_params=pltpu.CompilerParams(needs_layout_passes=False),
        scratch_types=[
            pltpu.VMEM((chunks, 128), jnp.int32),
            pltpu.VMEM((ppw * 4,), jnp.float32),
            pltpu.VMEM((_NBUF, rows_pc, _NFREQ), jnp.float32),
            pltpu.VMEM((_NOUT, _GROUP_PTS, _NFREQ), jnp.float32),
            pltpu.SemaphoreType.DMA,
            pltpu.SemaphoreType.DMA,
        ],
    )
    def sc_kernel(table_h, idx_h, wgt_h, out_h, idx_v, wgt_v, rows_v, outb,
                  gsem, fsem):
        wid = lax.axis_index("s") * _NUM_CORES + lax.axis_index("c")
        pltpu.sync_copy(idx_h.at[pl.ds(wid * chunks, chunks)], idx_v)
        pltpu.sync_copy(wgt_h.at[pl.ds(wid * ppw * 4, ppw * 4)], wgt_v)

        def gather(c):
            return pltpu.async_copy(
                table_h.at[idx_v.at[c]], rows_v.at[c % _NBUF], gsem)

        for c in range(_NBUF - 1):
            gather(c)

        def flush_copy(g):
            gstart = wid * ppw + g * _GROUP_PTS
            return pltpu.make_async_copy(
                outb.at[g % _NOUT], out_h.at[pl.ds(gstart, _GROUP_PTS)], fsem)

        def group_body(g, carry):
            @pl.when(g >= _NOUT)
            def _drain():
                flush_copy(g - _NOUT).wait()

            ob = outb.at[g % _NOUT]
            for cc in range(cpg):
                c = g * cpg + cc
                pltpu.make_async_copy(
                    table_h.at[idx_v.at[c]], rows_v.at[c % _NBUF], gsem).wait()

                @pl.when(c + _NBUF - 1 < chunks)
                def _prefetch():
                    gather(c + _NBUF - 1)

                rows = rows_v.at[c % _NBUF]

                @plsc.parallel_loop(0, _CHUNK_PTS, 1, unroll=4)
                def _pts(pp):
                    wof = 4 * (c * _CHUNK_PTS + pp)
                    accs = [None] * nblk
                    for k in range(4):
                        wv = plsc.load_gather(
                            wgt_v, [jnp.full((_LANES,), wof + k, jnp.int32)])
                        r = 4 * pp + k
                        for j in range(nblk):
                            term = wv * rows[r, pl.ds(j * _LANES, _LANES)]
                            accs[j] = term if k == 0 else accs[j] + term
                    prow = cc * _CHUNK_PTS + pp
                    for j in range(nblk):
                        ob[prow, pl.ds(j * _LANES, _LANES)] = accs[j]

            flush_copy(g).start()
            return carry

        lax.fori_loop(0, groups, group_body, 0)
        for g in range(groups - _NOUT, groups):
            flush_copy(g).wait()

    return sc_kernel(table, idx2d, wgt)


def kernel(params, inds, wgts, freqs):
    table = params.reshape(_NFREQ, _NPIX).T
    idx2d = inds.astype(jnp.int32).reshape(_NPTS * 4 // 128, 128)
    wgt = wgts.astype(jnp.float32).reshape(_NPTS * 4)
    out_pt = _pixel_beam_sc(table, idx2d, wgt)
    return out_pt.T.reshape(1, 1, 1, _NFREQ, _NPTS)
